# Initial kernel scaffold; baseline (speedup 1.0000x reference)
import jax
import jax.numpy as jnp
from jax.experimental import pallas as pl


def kernel(x, edge_index, W_enc, b_enc, ggnn_W, gru_W_ih, gru_W_hh, gru_b_ih, gru_b_hh, conv1_w, conv1_b, conv2_w, conv2_b, fc1_w, fc1_b, fc2_w, fc2_b):
    raise NotImplementedError("write your pallas kernel here")


# speedup vs baseline: 4.3396x; 4.3396x over previous
import functools

import jax
import jax.numpy as jnp
from jax import lax
from jax.experimental import pallas as pl
from jax.experimental.pallas import tpu as pltpu
from jax.experimental.pallas import tpu_sc as plsc

N_NODES = 10000
N_EDGES = 640000
F = 128
HF = 64
N_LAYERS = 6

NC, NS = 2, 16
EPT = N_EDGES // NS
CHUNK = 400
ROWB = 624
HROWB = ROWB // 2
REM = N_NODES - NS * ROWB


def _sc_scatter_body(m2_hbm, src_hbm, dst_hbm, out_hbm,
                     m_sh, agg_sh, src_v, dst_v, rows_v, sem):
    c = lax.axis_index("c")
    s = lax.axis_index("s")

    def _zrow(r, carry):
        for j in range(HF // 16):
            rows_v[r, pl.ds(j * 16, 16)] = jnp.zeros((16,), jnp.float32)
        return carry
    lax.fori_loop(0, HROWB, _zrow, 0)

    row0 = s * ROWB
    pltpu.sync_copy(m2_hbm.at[c, pl.ds(row0, ROWB)], m_sh.at[pl.ds(row0, ROWB)])
    zrows = rows_v.at[pl.ds(0, HROWB)]
    pltpu.sync_copy(zrows, agg_sh.at[pl.ds(row0, HROWB)])
    pltpu.sync_copy(zrows, agg_sh.at[pl.ds(row0 + HROWB, HROWB)])

    @pl.when(s == NS - 1)
    def _tail_in():
        t0 = NS * ROWB
        pltpu.sync_copy(m2_hbm.at[c, pl.ds(t0, REM)], m_sh.at[pl.ds(t0, REM)])
        pltpu.sync_copy(rows_v.at[pl.ds(0, REM)], agg_sh.at[pl.ds(t0, REM)])

    plsc.subcore_barrier()

    def _chunk(t, carry):
        base = s * EPT + t * CHUNK
        pltpu.sync_copy(src_hbm.at[pl.ds(base, CHUNK)], src_v)
        pltpu.sync_copy(dst_hbm.at[pl.ds(base, CHUNK)], dst_v)
        pltpu.async_copy(m_sh.at[src_v], rows_v, sem).wait()
        pltpu.sync_copy(rows_v, agg_sh.at[dst_v], add=True)
        return carry
    lax.fori_loop(0, EPT // CHUNK, _chunk, 0)

    plsc.subcore_barrier()
    pltpu.sync_copy(agg_sh.at[pl.ds(row0, ROWB)], out_hbm.at[c, pl.ds(row0, ROWB)])

    @pl.when(s == NS - 1)
    def _tail_out():
        t0 = NS * ROWB
        pltpu.sync_copy(agg_sh.at[pl.ds(t0, REM)], out_hbm.at[c, pl.ds(t0, REM)])


@functools.cache
def _sc_scatter_kernel():
    return functools.partial(
        pl.kernel,
        out_type=jax.ShapeDtypeStruct((NC, N_NODES, HF), jnp.float32),
        mesh=plsc.VectorSubcoreMesh(core_axis_name="c", subcore_axis_name="s",
                                    num_cores=NC, num_subcores=NS),
        scratch_types=[
            pltpu.VMEM_SHARED((N_NODES, HF), jnp.float32),
            pltpu.VMEM_SHARED((N_NODES, HF), jnp.float32),
            pltpu.VMEM((CHUNK,), jnp.int32),
            pltpu.VMEM((CHUNK,), jnp.int32),
            pltpu.VMEM((CHUNK, HF), jnp.float32),
            pltpu.SemaphoreType.DMA,
        ],
        compiler_params=pltpu.CompilerParams(use_tc_tiling_on_sc=False),
    )(_sc_scatter_body)


def _sc_scatter(m2, src, dst):
    return _sc_scatter_kernel()(m2, src, dst)


BR = 2000


def _prep_body(x_ref, wenc_ref, benc_ref, w0_ref, h_ref, m2_ref):
    h = jnp.maximum(
        jnp.dot(x_ref[...], wenc_ref[...], preferred_element_type=jnp.float32)
        + benc_ref[...], 0.0)
    h_ref[...] = h
    m = jnp.dot(h, w0_ref[...], preferred_element_type=jnp.float32)
    m2_ref[0] = m[:, :HF]
    m2_ref[1] = m[:, HF:]


def _gru_body(agg2_ref, h_ref, wih_ref, whh_ref, bih_ref, bhh_ref, wnext_ref,
              ho_ref, m2_ref):
    agg = jnp.concatenate([agg2_ref[0], agg2_ref[1]], axis=1)
    h = h_ref[...]
    gi = jnp.dot(agg, wih_ref[...], preferred_element_type=jnp.float32) + bih_ref[...]
    gh = jnp.dot(h, whh_ref[...], preferred_element_type=jnp.float32) + bhh_ref[...]
    r = jax.nn.sigmoid(gi[:, :F] + gh[:, :F])
    z = jax.nn.sigmoid(gi[:, F:2 * F] + gh[:, F:2 * F])
    n = jnp.tanh(gi[:, 2 * F:] + r * gh[:, 2 * F:])
    hn = (1.0 - z) * n + z * h
    ho_ref[...] = hn
    m = jnp.dot(hn, wnext_ref[...], preferred_element_type=jnp.float32)
    m2_ref[0] = m[:, :HF]
    m2_ref[1] = m[:, HF:]


BH = 80


def _conv_branch(cc, w1r, b1c, w2t, b2r, wfc, plen):
    bsz = cc.shape[0]
    zcol = jnp.zeros((bsz, 1), jnp.float32)
    ccext = jnp.concatenate([zcol, cc, zcol], axis=1)
    ccext3 = lax.broadcast_in_dim(ccext, (bsz, plen + 2, 50), (0, 1))
    y = b1c.T[None, :, :]
    for k in range(3):
        y = y + w1r.T[None, k:k + 1, :] * ccext3[:, k:k + plen, :]
    y = jnp.maximum(y, 0.0)
    p1 = plen // 2 - 1
    yr = y.reshape(bsz, plen // 2, 2, 50)
    e = yr[:, :, 0, :]
    o = yr[:, :, 1, :]
    pool1 = jnp.maximum(jnp.maximum(e[:, :p1, :], o[:, :p1, :]), e[:, 1:p1 + 1, :])
    t2 = lax.dot_general(pool1, w2t, (((2,), (0,)), ((), ())),
                         preferred_element_type=jnp.float32) + b2r
    p2 = (p1 - 2) // 2 + 1
    t2p = jnp.concatenate(
        [t2, jnp.zeros((bsz, 1, 20), jnp.float32)], axis=1)
    t2r = t2p.reshape(bsz, (p1 + 1) // 2, 2, 20)
    pool2 = jnp.maximum(t2r[:, :p2, 0, :], t2r[:, :p2, 1, :])
    return jnp.sum(pool2 * wfc[None, :, :], axis=(1, 2))


def _head_body(h_ref, x_ref, w1r_ref, b1c_ref, w2t_ref, b2r_ref,
               wz_ref, wy_ref, fb1_ref, fb2_ref, out_ref):
    h = h_ref[...]
    x = x_ref[...]
    w1r, b1c, w2t, b2r = w1r_ref[...], b1c_ref[...], w2t_ref[...], b2r_ref[...]
    cc = jnp.concatenate([h, x], axis=1)
    rz = _conv_branch(cc, w1r, b1c, w2t, b2r, wz_ref[...], 2 * F) + fb1_ref[0, 0]
    ry = _conv_branch(h, w1r, b1c, w2t, b2r, wy_ref[...], F) + fb2_ref[0, 0]
    res = jax.nn.sigmoid(rz * ry)
    out_ref[...] = jnp.broadcast_to(res[:, None], (BH, F))


def _full_spec(arr):
    nd = arr.ndim
    return pl.BlockSpec(arr.shape, lambda i, _nd=nd: (0,) * _nd)


def kernel(x, edge_index, W_enc, b_enc, ggnn_W, gru_W_ih, gru_W_hh, gru_b_ih,
           gru_b_hh, conv1_w, conv1_b, conv2_w, conv2_b, fc1_w, fc1_b,
           fc2_w, fc2_b):
    n = N_NODES
    wenc = jnp.pad(W_enc, ((0, 0), (0, F - W_enc.shape[1])))
    benc = jnp.pad(b_enc, (0, F - b_enc.shape[0]))[None, :]
    wih_t = gru_W_ih.T
    whh_t = gru_W_hh.T
    bih = gru_b_ih[None, :]
    bhh = gru_b_hh[None, :]
    w1r = conv1_w[:, 0, :]
    b1c = conv1_b[:, None]
    w2t = conv2_w[:, :, 0].T
    b2r = conv2_b[None, None, :]
    wz = fc1_w.reshape(20, 63).transpose(1, 0)
    wy = fc2_w.reshape(20, 31).transpose(1, 0)
    fb1 = fc1_b[None, :]
    fb2 = fc2_b[None, :]
    src = edge_index[0]
    dst = edge_index[1]

    grid = n // BR
    row_blk = pl.BlockSpec((BR, F), lambda i: (i, 0))
    m2_blk = pl.BlockSpec((NC, BR, HF), lambda i: (0, i, 0))

    h, m2 = pl.pallas_call(
        _prep_body,
        grid=(grid,),
        in_specs=[row_blk, _full_spec(wenc), _full_spec(benc),
                  _full_spec(ggnn_W[0])],
        out_specs=[row_blk, m2_blk],
        out_shape=[jax.ShapeDtypeStruct((n, F), jnp.float32),
                   jax.ShapeDtypeStruct((NC, n, HF), jnp.float32)],
    )(x, wenc, benc, ggnn_W[0])

    gru_call = pl.pallas_call(
        _gru_body,
        grid=(grid,),
        in_specs=[m2_blk, row_blk, _full_spec(wih_t), _full_spec(whh_t),
                  _full_spec(bih), _full_spec(bhh), _full_spec(ggnn_W[0])],
        out_specs=[row_blk, m2_blk],
        out_shape=[jax.ShapeDtypeStruct((n, F), jnp.float32),
                   jax.ShapeDtypeStruct((NC, n, HF), jnp.float32)],
    )

    for i in range(N_LAYERS):
        agg2 = _sc_scatter(m2, src, dst)
        wnext = ggnn_W[i + 1] if i + 1 < N_LAYERS else ggnn_W[0]
        h, m2 = gru_call(agg2, h, wih_t, whh_t, bih, bhh, wnext)

    head_grid = n // BH
    hb = pl.BlockSpec((BH, F), lambda i: (i, 0))
    out = pl.pallas_call(
        _head_body,
        grid=(head_grid,),
        in_specs=[hb, hb, _full_spec(w1r), _full_spec(b1c), _full_spec(w2t),
                  _full_spec(b2r), _full_spec(wz), _full_spec(wy),
                  _full_spec(fb1), _full_spec(fb2)],
        out_specs=hb,
        out_shape=jax.ShapeDtypeStruct((n, F), jnp.float32),
    )(h, x, w1r, b1c, w2t, b2r, wz, wy, fb1, fb2)

    return out[:, 0:1]


# speedup vs baseline: 6.4048x; 1.4759x over previous
import functools

import jax
import jax.numpy as jnp
from jax import lax
from jax.experimental import pallas as pl
from jax.experimental.pallas import tpu as pltpu
from jax.experimental.pallas import tpu_sc as plsc

N_NODES = 10000
N_EDGES = 640000
F = 128
HF = 64
N_LAYERS = 6

NC, NS = 2, 16
EPT = N_EDGES // NS
CHUNK = 200
NCH = 5
IB = NCH * CHUNK
NB = EPT // IB
ROWB = 624
REM = N_NODES - NS * ROWB


def _sc_scatter_body(m2_hbm, srcb_hbm, dstb_hbm, out_hbm,
                     m_sh, agg_sh, src_v, dst_v, rows_v, sem_i, sem_g):
    c = lax.axis_index("c")
    s = lax.axis_index("s")
    row0 = s * ROWB

    mcopy = pltpu.async_copy(m2_hbm.at[c, pl.ds(row0, ROWB)],
                             m_sh.at[pl.ds(row0, ROWB)], sem_i)

    def _zrow(r, carry):
        for j in range(HF // 16):
            rows_v[0, r, pl.ds(j * 16, 16)] = jnp.zeros((16,), jnp.float32)
        return carry
    lax.fori_loop(0, CHUNK, _zrow, 0)

    for off in range(0, ROWB - CHUNK + 1, CHUNK):
        pltpu.sync_copy(rows_v.at[0], agg_sh.at[pl.ds(row0 + off, CHUNK)])
    rem0 = ROWB - (ROWB // CHUNK) * CHUNK
    if rem0:
        pltpu.sync_copy(rows_v.at[0, pl.ds(0, rem0)],
                        agg_sh.at[pl.ds(row0 + ROWB - rem0, rem0)])

    @pl.when(s == NS - 1)
    def _tail_in():
        t0 = NS * ROWB
        pltpu.sync_copy(m2_hbm.at[c, pl.ds(t0, REM)], m_sh.at[pl.ds(t0, REM)])
        pltpu.sync_copy(rows_v.at[0, pl.ds(0, REM)], agg_sh.at[pl.ds(t0, REM)])

    mcopy.wait()
    plsc.subcore_barrier()

    brow0 = s * (EPT // CHUNK)
    pltpu.async_copy(srcb_hbm.at[pl.ds(brow0, NCH)], src_v.at[0], sem_i)
    pltpu.async_copy(dstb_hbm.at[pl.ds(brow0, NCH)], dst_v.at[0], sem_i)

    def _block(bi, pb):
        brow = brow0 + bi * NCH
        pltpu.make_async_copy(srcb_hbm.at[pl.ds(brow, NCH)],
                              src_v.at[pb], sem_i).wait()
        pltpu.make_async_copy(dstb_hbm.at[pl.ds(brow, NCH)],
                              dst_v.at[pb], sem_i).wait()

        @pl.when(bi + 1 < NB)
        def _prefetch():
            nrow = brow + NCH
            pltpu.async_copy(srcb_hbm.at[pl.ds(nrow, NCH)],
                             src_v.at[1 - pb], sem_i)
            pltpu.async_copy(dstb_hbm.at[pl.ds(nrow, NCH)],
                             dst_v.at[1 - pb], sem_i)

        gh = pltpu.async_copy(m_sh.at[src_v.at[pb, 0]], rows_v.at[0], sem_g)
        for j in range(NCH):
            gh.wait()
            if j + 1 < NCH:
                gh = pltpu.async_copy(m_sh.at[src_v.at[pb, j + 1]],
                                      rows_v.at[(j + 1) % 2], sem_g)
            pltpu.sync_copy(rows_v.at[j % 2], agg_sh.at[dst_v.at[pb, j]],
                            add=True)

    def _pair(i, carry):
        _block(2 * i, 0)
        _block(2 * i + 1, 1)
        return carry
    lax.fori_loop(0, NB // 2, _pair, 0)

    plsc.subcore_barrier()
    pltpu.sync_copy(agg_sh.at[pl.ds(row0, ROWB)], out_hbm.at[c, pl.ds(row0, ROWB)])

    @pl.when(s == NS - 1)
    def _tail_out():
        t0 = NS * ROWB
        pltpu.sync_copy(agg_sh.at[pl.ds(t0, REM)], out_hbm.at[c, pl.ds(t0, REM)])


@functools.cache
def _sc_scatter_kernel():
    return functools.partial(
        pl.kernel,
        out_type=jax.ShapeDtypeStruct((NC, N_NODES, HF), jnp.float32),
        mesh=plsc.VectorSubcoreMesh(core_axis_name="c", subcore_axis_name="s",
                                    num_cores=NC, num_subcores=NS),
        scratch_types=[
            pltpu.VMEM_SHARED((N_NODES, HF), jnp.float32),
            pltpu.VMEM_SHARED((N_NODES, HF), jnp.float32),
            pltpu.VMEM((2, NCH, CHUNK), jnp.int32),
            pltpu.VMEM((2, NCH, CHUNK), jnp.int32),
            pltpu.VMEM((2, CHUNK, HF), jnp.float32),
            pltpu.SemaphoreType.DMA,
            pltpu.SemaphoreType.DMA,
        ],
        compiler_params=pltpu.CompilerParams(use_tc_tiling_on_sc=False),
    )(_sc_scatter_body)


def _sc_scatter(m2, src, dst):
    return _sc_scatter_kernel()(m2, src, dst)


BR = 2000


def _prep_body(x_ref, wenc_ref, benc_ref, w0_ref, h_ref, m2_ref):
    h = jnp.maximum(
        jnp.dot(x_ref[...], wenc_ref[...], preferred_element_type=jnp.float32)
        + benc_ref[...], 0.0)
    h_ref[...] = h
    m = jnp.dot(h, w0_ref[...], preferred_element_type=jnp.float32)
    m2_ref[0] = m[:, :HF]
    m2_ref[1] = m[:, HF:]


def _gru_body(agg2_ref, h_ref, wih_ref, whh_ref, bih_ref, bhh_ref, wnext_ref,
              ho_ref, m2_ref):
    agg = jnp.concatenate([agg2_ref[0], agg2_ref[1]], axis=1)
    h = h_ref[...]
    gi = jnp.dot(agg, wih_ref[...], preferred_element_type=jnp.float32) + bih_ref[...]
    gh = jnp.dot(h, whh_ref[...], preferred_element_type=jnp.float32) + bhh_ref[...]
    r = jax.nn.sigmoid(gi[:, :F] + gh[:, :F])
    z = jax.nn.sigmoid(gi[:, F:2 * F] + gh[:, F:2 * F])
    n = jnp.tanh(gi[:, 2 * F:] + r * gh[:, 2 * F:])
    hn = (1.0 - z) * n + z * h
    ho_ref[...] = hn
    m = jnp.dot(hn, wnext_ref[...], preferred_element_type=jnp.float32)
    m2_ref[0] = m[:, :HF]
    m2_ref[1] = m[:, HF:]


BH = 200


def _conv_branch(taps, w1c, b1c, w2, b2c, wfc):
    r3m, r0, r1, r2, r3, r0p = [t[None, :, :] for t in taps]
    a = r0.shape[2]
    w1 = [w1c[k] for k in range(3)]
    y0 = jnp.maximum(w1[0] * r3m + w1[1] * r0 + w1[2] * r1 + b1c, 0.0)
    y1 = jnp.maximum(w1[0] * r0 + w1[1] * r1 + w1[2] * r2 + b1c, 0.0)
    y2 = jnp.maximum(w1[0] * r1 + w1[1] * r2 + w1[2] * r3 + b1c, 0.0)
    y3 = jnp.maximum(w1[0] * r2 + w1[1] * r3 + w1[2] * r0p + b1c, 0.0)
    pe = jnp.maximum(jnp.maximum(y0, y1), y2)
    po = jnp.maximum(jnp.maximum(y2[:, :, :a - 1], y3[:, :, :a - 1]),
                     y0[:, :, 1:a])
    t2e = lax.dot_general(w2, pe, (((1,), (0,)), ((), ())),
                          preferred_element_type=jnp.float32) + b2c
    t2o = lax.dot_general(w2, po, (((1,), (0,)), ((), ())),
                          preferred_element_type=jnp.float32) + b2c
    pool2 = jnp.maximum(t2e[:, :, :a - 1], t2o)
    return jnp.sum(jnp.sum(pool2 * wfc, axis=0), axis=1)


def _head_body(z0_ref, z1_ref, z2_ref, z3_ref, z3m_ref, z0p_ref,
               y0_ref, y1_ref, y2_ref, y3_ref, y3m_ref, y0p_ref,
               w1c_ref, b1c_ref, w2_ref, b2c_ref,
               wz_ref, wy_ref, fb1_ref, fb2_ref, out_ref):
    w1c, b1c, w2, b2c = w1c_ref[...], b1c_ref[...], w2_ref[...], b2c_ref[...]
    ztaps = (z3m_ref[...], z0_ref[...], z1_ref[...], z2_ref[...],
             z3_ref[...], z0p_ref[...])
    ytaps = (y3m_ref[...], y0_ref[...], y1_ref[...], y2_ref[...],
             y3_ref[...], y0p_ref[...])
    rz = _conv_branch(ztaps, w1c, b1c, w2, b2c, wz_ref[...]) + fb1_ref[0, 0]
    ry = _conv_branch(ytaps, w1c, b1c, w2, b2c, wy_ref[...]) + fb2_ref[0, 0]
    res = jax.nn.sigmoid(rz * ry)
    out_ref[...] = jnp.broadcast_to(res[:, None], (BH, F))


def _full_spec(arr):
    nd = arr.ndim
    return pl.BlockSpec(arr.shape, lambda i, _nd=nd: (0,) * _nd)


def kernel(x, edge_index, W_enc, b_enc, ggnn_W, gru_W_ih, gru_W_hh, gru_b_ih,
           gru_b_hh, conv1_w, conv1_b, conv2_w, conv2_b, fc1_w, fc1_b,
           fc2_w, fc2_b):
    n = N_NODES
    wenc = jnp.pad(W_enc, ((0, 0), (0, F - W_enc.shape[1])))
    benc = jnp.pad(b_enc, (0, F - b_enc.shape[0]))[None, :]
    wih_t = gru_W_ih.T
    whh_t = gru_W_hh.T
    bih = gru_b_ih[None, :]
    bhh = gru_b_hh[None, :]
    w1c = conv1_w[:, 0, :].T.reshape(3, 50, 1, 1)
    b1c = conv1_b[:, None, None]
    w2 = conv2_w[:, :, 0]
    b2c = conv2_b[:, None, None]
    wz3 = fc1_w.reshape(20, 63)[:, None, :]
    wy3 = fc2_w.reshape(20, 31)[:, None, :]
    fb1 = fc1_b[None, :]
    fb2 = fc2_b[None, :]
    src = edge_index[0].reshape(N_EDGES // CHUNK, CHUNK)
    dst = edge_index[1].reshape(N_EDGES // CHUNK, CHUNK)

    grid = n // BR
    row_blk = pl.BlockSpec((BR, F), lambda i: (i, 0))
    m2_blk = pl.BlockSpec((NC, BR, HF), lambda i: (0, i, 0))

    h, m2 = pl.pallas_call(
        _prep_body,
        grid=(grid,),
        in_specs=[row_blk, _full_spec(wenc), _full_spec(benc),
                  _full_spec(ggnn_W[0])],
        out_specs=[row_blk, m2_blk],
        out_shape=[jax.ShapeDtypeStruct((n, F), jnp.float32),
                   jax.ShapeDtypeStruct((NC, n, HF), jnp.float32)],
    )(x, wenc, benc, ggnn_W[0])

    gru_call = pl.pallas_call(
        _gru_body,
        grid=(grid,),
        in_specs=[m2_blk, row_blk, _full_spec(wih_t), _full_spec(whh_t),
                  _full_spec(bih), _full_spec(bhh), _full_spec(ggnn_W[0])],
        out_specs=[row_blk, m2_blk],
        out_shape=[jax.ShapeDtypeStruct((n, F), jnp.float32),
                   jax.ShapeDtypeStruct((NC, n, HF), jnp.float32)],
    )

    for i in range(N_LAYERS):
        agg2 = _sc_scatter(m2, src, dst)
        wnext = ggnn_W[i + 1] if i + 1 < N_LAYERS else ggnn_W[0]
        h, m2 = gru_call(agg2, h, wih_t, whh_t, bih, bhh, wnext)

    def _residues(arr):
        rs = [arr[:, r::4] for r in range(4)]
        zc = jnp.zeros((n, 1), jnp.float32)
        r3m = jnp.concatenate([zc, rs[3][:, :-1]], axis=1)
        r0p = jnp.concatenate([rs[0][:, 1:], zc], axis=1)
        return rs + [r3m, r0p]

    cc = jnp.concatenate([h, x], axis=1)
    zres = _residues(cc)
    yres = _residues(h)

    head_grid = n // BH
    zb = pl.BlockSpec((BH, 2 * F // 4), lambda i: (i, 0))
    yb = pl.BlockSpec((BH, F // 4), lambda i: (i, 0))
    hb = pl.BlockSpec((BH, F), lambda i: (i, 0))
    out = pl.pallas_call(
        _head_body,
        grid=(head_grid,),
        in_specs=[zb] * 6 + [yb] * 6 + [
            _full_spec(w1c), _full_spec(b1c), _full_spec(w2),
            _full_spec(b2c), _full_spec(wz3), _full_spec(wy3),
            _full_spec(fb1), _full_spec(fb2)],
        out_specs=hb,
        out_shape=jax.ShapeDtypeStruct((n, F), jnp.float32),
    )(zres[0], zres[1], zres[2], zres[3], zres[4], zres[5],
      yres[0], yres[1], yres[2], yres[3], yres[4], yres[5],
      w1c, b1c, w2, b2c, wz3, wy3, fb1, fb2)

    return out[:, 0:1]
